# baseline (speedup 1.0000x reference)
import functools

import jax
import jax.numpy as jnp
from jax import lax
from jax.experimental import pallas as pl
from jax.experimental.pallas import tpu as pltpu
from jax.experimental.pallas import tpu_sc as plsc

B = 16384
F = 26
V = 100000
D = 16
ADAPT = 128

NC = 2
NS = 16
NW = NC * NS
K = F * D
RPW = K // NW
BH = B // 2


UNROLL = 16


def _rowsweep_body_impl(tab_hbm, xt_hbm, out_hbm, row_v, x_v, o_v, rpw):
    wid = lax.axis_index("s") * NC + lax.axis_index("c")
    k0 = wid * rpw

    def row_step(j, carry):
        k = k0 + j
        f = k // D
        d = k % D

        @pl.when(jnp.logical_or(j == 0, f != (k - 1) // D))
        def _():
            pltpu.sync_copy(xt_hbm.at[f], x_v)

        pltpu.sync_copy(tab_hbm.at[f, d], row_v)

        def half_step(h):
            base = h * BH

            def body(i, c):
                for u in range(UNROLL):
                    off = (i * UNROLL + u) * 16
                    idx = x_v[pl.ds(base + off, 16)]
                    o_v[pl.ds(off, 16)] = plsc.load_gather(row_v, [idx])
                return c

            lax.fori_loop(0, BH // (16 * UNROLL), body, 0)
            pltpu.sync_copy(o_v, out_hbm.at[k, pl.ds(base, BH)])

        half_step(0)
        half_step(1)
        return carry

    lax.fori_loop(0, rpw, row_step, 0)


def _make_rowsweep(nf):
    rpw = nf * D // NW

    def body(tab_hbm, xt_hbm, out_hbm, row_v, x_v, o_v):
        _rowsweep_body_impl(tab_hbm, xt_hbm, out_hbm, row_v, x_v, o_v, rpw)

    mesh = plsc.VectorSubcoreMesh(core_axis_name="c", subcore_axis_name="s")
    return pl.kernel(
        body,
        out_type=jax.ShapeDtypeStruct((nf * D, B), jnp.float32),
        mesh=mesh,
        scratch_types=[
            pltpu.VMEM((V,), jnp.float32),
            pltpu.VMEM((B,), jnp.int32),
            pltpu.VMEM((BH,), jnp.float32),
        ],
        compiler_params=pltpu.CompilerParams(use_tc_tiling_on_sc=True,
                                             needs_layout_passes=False),
    )


F1 = 16
K1 = F1 * D
K2 = K - K1
BLK = 2048


def _dense_a_body(ft_ref, lw_ref, h_ref):
    h_ref[...] = lax.dot_general(ft_ref[...], lw_ref[...],
                                 (((0,), (0,)), ((), ())),
                                 preferred_element_type=jnp.float32)


def _tc_dense_a(ft1, lw1):
    return pl.pallas_call(
        _dense_a_body,
        grid=(B // BLK,),
        in_specs=[
            pl.BlockSpec((K1, BLK), lambda i: (0, i)),
            pl.BlockSpec((K1, ADAPT), lambda i: (0, 0)),
        ],
        out_specs=pl.BlockSpec((BLK, ADAPT), lambda i: (i, 0)),
        out_shape=jax.ShapeDtypeStruct((B, ADAPT), jnp.float32),
    )(ft1, lw1)


def _dense_b_body(ft_ref, hp_ref, lw_ref, lb_ref, w1_ref, b1_ref, w2_ref,
                  b2_ref, out_ref):
    h = lax.dot_general(ft_ref[...], lw_ref[...], (((0,), (0,)), ((), ())),
                        preferred_element_type=jnp.float32)
    h = h + hp_ref[...] + lb_ref[...]
    h = jnp.maximum(jnp.dot(h, w1_ref[...],
                            preferred_element_type=jnp.float32) + b1_ref[...],
                    0.0)
    out_ref[...] = jnp.dot(h, w2_ref[...],
                           preferred_element_type=jnp.float32) + b2_ref[...]


def _tc_dense_b(ft2, h_part, lw2, lin_b, w1, b1, w2, b2):
    return pl.pallas_call(
        _dense_b_body,
        grid=(B // BLK,),
        in_specs=[
            pl.BlockSpec((K2, BLK), lambda i: (0, i)),
            pl.BlockSpec((BLK, ADAPT), lambda i: (i, 0)),
            pl.BlockSpec((K2, ADAPT), lambda i: (0, 0)),
            pl.BlockSpec((1, ADAPT), lambda i: (0, 0)),
            pl.BlockSpec((ADAPT, ADAPT), lambda i: (0, 0)),
            pl.BlockSpec((1, ADAPT), lambda i: (0, 0)),
            pl.BlockSpec((ADAPT, 1), lambda i: (0, 0)),
            pl.BlockSpec((1, 1), lambda i: (0, 0)),
        ],
        out_specs=pl.BlockSpec((BLK, 1), lambda i: (i, 0)),
        out_shape=jax.ShapeDtypeStruct((B, 1), jnp.float32),
    )(ft2, h_part, lw2, lin_b, w1, b1, w2, b2)


@jax.jit
def kernel(x, emb_tables, lin_w, lin_b, w1, b1, w2, b2):
    tab3 = jnp.transpose(emb_tables, (0, 2, 1))
    xt = x.T
    ft1 = _make_rowsweep(F1)(tab3[:F1], xt[:F1])
    ft2 = _make_rowsweep(F - F1)(tab3[F1:], xt[F1:])
    h_part = _tc_dense_a(ft1, lin_w[:K1])
    return _tc_dense_b(ft2, h_part, lin_w[K1:], lin_b.reshape(1, ADAPT), w1,
                       b1.reshape(1, ADAPT), w2, b2.reshape(1, 1))


# speedup vs baseline: 1.6105x; 1.6105x over previous
import functools

import jax
import jax.numpy as jnp
from jax import lax
from jax.experimental import pallas as pl
from jax.experimental.pallas import tpu as pltpu
from jax.experimental.pallas import tpu_sc as plsc

B = 16384
F = 26
V = 100000
D = 16
ADAPT = 128

NC = 2
NS = 16
NW = NC * NS
K = F * D
RPW = K // NW
BH = B // 2


UNROLL = 16


BQ = B // 4


def _rowsweep_body(tab_hbm, xt_hbm, out_hbm, row_v, x_v, oa_v, ob_v, sa, sb):
    wid = lax.axis_index("s") * NC + lax.axis_index("c")
    k0 = wid * RPW
    obufs = (oa_v, ob_v)
    sems = (sa, sb)

    def wait_store(buf, sem):
        pltpu.make_async_copy(buf, out_hbm.at[0, pl.ds(0, BQ)], sem).wait()

    def row_step(j, carry):
        k = k0 + j
        f = k // D
        d = k % D

        @pl.when(jnp.logical_or(j == 0, f != (k - 1) // D))
        def _():
            pltpu.sync_copy(xt_hbm.at[f], x_v)

        pltpu.sync_copy(tab_hbm.at[f, d], row_v)

        for q in range(4):
            buf = obufs[q % 2]
            sem = sems[q % 2]
            if q < 2:
                @pl.when(j > 0)
                def _():
                    wait_store(buf, sem)
            else:
                wait_store(buf, sem)

            def body(i, c, q=q, buf=buf):
                for u in range(UNROLL):
                    off = (i * UNROLL + u) * 16
                    idx = x_v[pl.ds(q * BQ + off, 16)]
                    buf[pl.ds(off, 16)] = plsc.load_gather(row_v, [idx])
                return c

            lax.fori_loop(0, BQ // (16 * UNROLL), body, 0)
            pltpu.async_copy(buf, out_hbm.at[k, pl.ds(q * BQ, BQ)], sem)
        return carry

    lax.fori_loop(0, RPW, row_step, 0)
    wait_store(oa_v, sa)
    wait_store(ob_v, sb)


@jax.jit
def _sc_rowsweep(tab3, xt):
    mesh = plsc.VectorSubcoreMesh(core_axis_name="c", subcore_axis_name="s")
    return pl.kernel(
        _rowsweep_body,
        out_type=jax.ShapeDtypeStruct((K, B), jnp.float32),
        mesh=mesh,
        scratch_types=[
            pltpu.VMEM((V,), jnp.float32),
            pltpu.VMEM((B,), jnp.int32),
            pltpu.VMEM((BQ,), jnp.float32),
            pltpu.VMEM((BQ,), jnp.float32),
            pltpu.SemaphoreType.DMA,
            pltpu.SemaphoreType.DMA,
        ],
        compiler_params=pltpu.CompilerParams(use_tc_tiling_on_sc=True,
                                             needs_layout_passes=False),
    )(tab3, xt)


def _dense_body(ft_ref, lw_ref, lb_ref, w1_ref, b1_ref, w2_ref, b2_ref,
                out_ref):
    h = lax.dot_general(ft_ref[...], lw_ref[...], (((0,), (0,)), ((), ())),
                        preferred_element_type=jnp.float32) + lb_ref[...]
    h = jnp.maximum(jnp.dot(h, w1_ref[...],
                            preferred_element_type=jnp.float32) + b1_ref[...],
                    0.0)
    out_ref[...] = jnp.dot(h, w2_ref[...],
                           preferred_element_type=jnp.float32) + b2_ref[...]


BLK = 2048


def _tc_dense(featT, lin_w, lin_b, w1, b1, w2, b2):
    grid = (B // BLK,)
    return pl.pallas_call(
        _dense_body,
        grid=grid,
        in_specs=[
            pl.BlockSpec((K, BLK), lambda i: (0, i)),
            pl.BlockSpec((K, ADAPT), lambda i: (0, 0)),
            pl.BlockSpec((1, ADAPT), lambda i: (0, 0)),
            pl.BlockSpec((ADAPT, ADAPT), lambda i: (0, 0)),
            pl.BlockSpec((1, ADAPT), lambda i: (0, 0)),
            pl.BlockSpec((ADAPT, 1), lambda i: (0, 0)),
            pl.BlockSpec((1, 1), lambda i: (0, 0)),
        ],
        out_specs=pl.BlockSpec((BLK, 1), lambda i: (i, 0)),
        out_shape=jax.ShapeDtypeStruct((B, 1), jnp.float32),
    )(featT, lin_w, lin_b, w1, b1, w2, b2)


def kernel(x, emb_tables, lin_w, lin_b, w1, b1, w2, b2):
    tab3 = jnp.transpose(emb_tables, (0, 2, 1))
    xt = x.T
    featT = _sc_rowsweep(tab3, xt)
    return _tc_dense(featT, lin_w, lin_b.reshape(1, ADAPT), w1,
                     b1.reshape(1, ADAPT), w2, b2.reshape(1, 1))


# speedup vs baseline: 1.6564x; 1.0286x over previous
import functools

import jax
import jax.numpy as jnp
from jax import lax
from jax.experimental import pallas as pl
from jax.experimental.pallas import tpu as pltpu
from jax.experimental.pallas import tpu_sc as plsc

B = 16384
F = 26
V = 100000
D = 16
ADAPT = 128

NC = 2
NS = 16
NW = NC * NS
K = F * D
RPW = K // NW
BH = B // 2


UNROLL = 16


BQ = B // 4


def _rowsweep_body(tab_hbm, xt_hbm, out_hbm, row_v, x_v, oa_v, ob_v, sa, sb):
    wid = lax.axis_index("s") * NC + lax.axis_index("c")
    k0 = wid * RPW
    obufs = (oa_v, ob_v)
    sems = (sa, sb)

    def wait_store(buf, sem):
        pltpu.make_async_copy(buf, out_hbm.at[0, pl.ds(0, BQ)], sem).wait()

    def row_step(j, carry):
        k = k0 + j
        f = k // D
        d = k % D

        @pl.when(jnp.logical_or(j == 0, f != (k - 1) // D))
        def _():
            pltpu.sync_copy(xt_hbm.at[f], x_v)

        pltpu.sync_copy(tab_hbm.at[f, d], row_v)

        for q in range(4):
            buf = obufs[q % 2]
            sem = sems[q % 2]
            if q < 2:
                @pl.when(j > 0)
                def _():
                    wait_store(buf, sem)
            else:
                wait_store(buf, sem)

            def body(i, c, q=q, buf=buf):
                for u in range(UNROLL):
                    off = (i * UNROLL + u) * 16
                    idx = x_v[pl.ds(q * BQ + off, 16)]
                    buf[pl.ds(off, 16)] = plsc.load_gather(row_v, [idx])
                return c

            lax.fori_loop(0, BQ // (16 * UNROLL), body, 0)
            pltpu.async_copy(buf, out_hbm.at[k, pl.ds(q * BQ, BQ)], sem)
        return carry

    lax.fori_loop(0, RPW, row_step, 0)
    wait_store(oa_v, sa)
    wait_store(ob_v, sb)


@jax.jit
def _sc_rowsweep(tab3, xt):
    mesh = plsc.VectorSubcoreMesh(core_axis_name="c", subcore_axis_name="s")
    return pl.kernel(
        _rowsweep_body,
        out_type=jax.ShapeDtypeStruct((K, B), jnp.float32),
        mesh=mesh,
        scratch_types=[
            pltpu.VMEM((V,), jnp.float32),
            pltpu.VMEM((B,), jnp.int32),
            pltpu.VMEM((BQ,), jnp.float32),
            pltpu.VMEM((BQ,), jnp.float32),
            pltpu.SemaphoreType.DMA,
            pltpu.SemaphoreType.DMA,
        ],
        compiler_params=pltpu.CompilerParams(use_tc_tiling_on_sc=True,
                                             needs_layout_passes=False),
    )(tab3, xt)


def _dense_body(ft_ref, lw_ref, lb_ref, w1_ref, b1_ref, w2_ref, b2_ref,
                out_ref):
    h = lax.dot_general(ft_ref[...], lw_ref[...], (((0,), (0,)), ((), ())),
                        preferred_element_type=jnp.float32) + lb_ref[...]
    h = jnp.maximum(jnp.dot(h, w1_ref[...],
                            preferred_element_type=jnp.float32) + b1_ref[...],
                    0.0)
    out_ref[...] = lax.dot_general(
        w2_ref[...], h, (((0,), (1,)), ((), ())),
        preferred_element_type=jnp.float32) + b2_ref[...]


BLK = 4096


def _tc_dense(featT, lin_w, lin_b, w1, b1, w2, b2):
    grid = (B // BLK,)
    return pl.pallas_call(
        _dense_body,
        grid=grid,
        in_specs=[
            pl.BlockSpec((K, BLK), lambda i: (0, i)),
            pl.BlockSpec((K, ADAPT), lambda i: (0, 0)),
            pl.BlockSpec((1, ADAPT), lambda i: (0, 0)),
            pl.BlockSpec((ADAPT, ADAPT), lambda i: (0, 0)),
            pl.BlockSpec((1, ADAPT), lambda i: (0, 0)),
            pl.BlockSpec((ADAPT, 1), lambda i: (0, 0)),
            pl.BlockSpec((1, 1), lambda i: (0, 0)),
        ],
        out_specs=pl.BlockSpec((1, BLK), lambda i: (0, i)),
        out_shape=jax.ShapeDtypeStruct((1, B), jnp.float32),
        compiler_params=pltpu.CompilerParams(
            fuse_transposed_lhs_in_matmul=True),
    )(featT, lin_w, lin_b, w1, b1, w2, b2)


def kernel(x, emb_tables, lin_w, lin_b, w1, b1, w2, b2):
    tab3 = jnp.transpose(emb_tables, (0, 2, 1))
    xt = x.T
    featT = _sc_rowsweep(tab3, xt)
    outT = _tc_dense(featT, lin_w, lin_b.reshape(1, ADAPT), w1,
                     b1.reshape(1, ADAPT), w2, b2.reshape(1, 1))
    return outT.T


# speedup vs baseline: 2.1528x; 1.2996x over previous
import functools

import jax
import jax.numpy as jnp
from jax import lax
from jax.experimental import pallas as pl
from jax.experimental.pallas import tpu as pltpu
from jax.experimental.pallas import tpu_sc as plsc

B = 16384
F = 26
V = 100000
D = 16
ADAPT = 128

NC = 2
NS = 16
NW = NC * NS
K = F * D
RPW = K // NW
BH = B // 2


UNROLL = 16


BQ = B // 4


def _rowsweep_body(tab_hbm, xt_hbm, out_hbm, row_v, x_v, oa_v, ob_v, sa, sb):
    wid = lax.axis_index("s") * NC + lax.axis_index("c")
    k0 = wid * RPW
    obufs = (oa_v, ob_v)
    sems = (sa, sb)

    def wait_store(buf, sem):
        pltpu.make_async_copy(buf, out_hbm.at[0, pl.ds(0, BQ)], sem).wait()

    def row_step(j, carry):
        k = k0 + j
        f = k // D
        d = k % D

        @pl.when(jnp.logical_or(j == 0, f != (k - 1) // D))
        def _():
            pltpu.sync_copy(xt_hbm.at[f], x_v)

        pltpu.sync_copy(tab_hbm.at[f, d], row_v)

        for q in range(4):
            buf = obufs[q % 2]
            sem = sems[q % 2]
            if q < 2:
                @pl.when(j > 0)
                def _():
                    wait_store(buf, sem)
            else:
                wait_store(buf, sem)

            @plsc.parallel_loop(0, BQ // 16, unroll=UNROLL)
            def _(i, q=q, buf=buf):
                off = i * 16
                idx = x_v[pl.ds(q * BQ + off, 16)]
                buf[pl.ds(off, 16)] = plsc.load_gather(row_v, [idx])
            pltpu.async_copy(buf, out_hbm.at[k, pl.ds(q * BQ, BQ)], sem)
        return carry

    lax.fori_loop(0, RPW, row_step, 0)
    wait_store(oa_v, sa)
    wait_store(ob_v, sb)


@jax.jit
def _sc_rowsweep(tab3, xt):
    mesh = plsc.VectorSubcoreMesh(core_axis_name="c", subcore_axis_name="s")
    return pl.kernel(
        _rowsweep_body,
        out_type=jax.ShapeDtypeStruct((K, B), jnp.float32),
        mesh=mesh,
        scratch_types=[
            pltpu.VMEM((V,), jnp.float32),
            pltpu.VMEM((B,), jnp.int32),
            pltpu.VMEM((BQ,), jnp.float32),
            pltpu.VMEM((BQ,), jnp.float32),
            pltpu.SemaphoreType.DMA,
            pltpu.SemaphoreType.DMA,
        ],
        compiler_params=pltpu.CompilerParams(use_tc_tiling_on_sc=True,
                                             needs_layout_passes=False),
    )(tab3, xt)


def _dense_body(ft_ref, lw_ref, lb_ref, w1_ref, b1_ref, w2_ref, b2_ref,
                out_ref):
    h = lax.dot_general(ft_ref[...], lw_ref[...], (((0,), (0,)), ((), ())),
                        preferred_element_type=jnp.float32) + lb_ref[...]
    h = jnp.maximum(jnp.dot(h, w1_ref[...],
                            preferred_element_type=jnp.float32) + b1_ref[...],
                    0.0)
    out_ref[...] = lax.dot_general(
        w2_ref[...], h, (((0,), (1,)), ((), ())),
        preferred_element_type=jnp.float32) + b2_ref[...]


BLK = 4096


def _tc_dense(featT, lin_w, lin_b, w1, b1, w2, b2):
    grid = (B // BLK,)
    return pl.pallas_call(
        _dense_body,
        grid=grid,
        in_specs=[
            pl.BlockSpec((K, BLK), lambda i: (0, i)),
            pl.BlockSpec((K, ADAPT), lambda i: (0, 0)),
            pl.BlockSpec((1, ADAPT), lambda i: (0, 0)),
            pl.BlockSpec((ADAPT, ADAPT), lambda i: (0, 0)),
            pl.BlockSpec((1, ADAPT), lambda i: (0, 0)),
            pl.BlockSpec((ADAPT, 1), lambda i: (0, 0)),
            pl.BlockSpec((1, 1), lambda i: (0, 0)),
        ],
        out_specs=pl.BlockSpec((1, BLK), lambda i: (0, i)),
        out_shape=jax.ShapeDtypeStruct((1, B), jnp.float32),
        compiler_params=pltpu.CompilerParams(
            fuse_transposed_lhs_in_matmul=True),
    )(featT, lin_w, lin_b, w1, b1, w2, b2)


def kernel(x, emb_tables, lin_w, lin_b, w1, b1, w2, b2):
    tab3 = jnp.transpose(emb_tables, (0, 2, 1))
    xt = x.T
    featT = _sc_rowsweep(tab3, xt)
    outT = _tc_dense(featT, lin_w, lin_b.reshape(1, ADAPT), w1,
                     b1.reshape(1, ADAPT), w2, b2.reshape(1, 1))
    return outT.T
